# baseline (speedup 1.0000x reference)
import functools

import jax
import jax.numpy as jnp
from jax import lax
from jax.experimental import pallas as pl
from jax.experimental.pallas import tpu as pltpu
from jax.experimental.pallas import tpu_sc as plsc

_L = 16
_T = 2 ** 19
_F = 2
_N_MIN = 16
_N_MAX = 4096
_MASK = _T - 1
_PI1 = -1640531535
_PI2 = 805459861

_LANES = 16
_NC = 2
_NS = 16
_NW = _NC * _NS
_RW = 16


def _build(N):
    P = N // _NW
    SB = 4
    Q = P // SB
    GQ = Q // _LANES
    NIQ = 8 * Q
    CH = 128
    NCH = NIQ // CH
    KOUT = 8

    mesh = plsc.VectorSubcoreMesh(core_axis_name="c", subcore_axis_name="s")

    @functools.partial(
        pl.kernel,
        out_type=jax.ShapeDtypeStruct((N, 2 * _L), jnp.float32),
        mesh=mesh,
        compiler_params=pltpu.CompilerParams(needs_layout_passes=False,
                                             use_tc_tiling_on_sc=False),
        scratch_types=[
            pltpu.VMEM((3 * P,), jnp.float32),
            pltpu.VMEM((3 * Q,), jnp.float32),
            pltpu.VMEM((_L * _LANES,), jnp.float32),
            pltpu.VMEM((NIQ,), jnp.int32),
            pltpu.VMEM((NIQ,), jnp.int32),
            pltpu.VMEM((NIQ, _RW), jnp.float32),
            pltpu.VMEM((Q, 2 * _L), jnp.float32),
            pltpu.SemaphoreType.DMA,
        ],
    )
    def enc(x_hbm, nl_hbm, table_hbm, out_hbm,
            xs_v, fr_v, nl_v, idx_v, col_v, rows_v, o_v, sem):
        wid = lax.axis_index("s") * _NC + lax.axis_index("c")
        base = wid * P
        pltpu.sync_copy(x_hbm.at[pl.ds(3 * base, 3 * P)], xs_v)
        pltpu.sync_copy(nl_hbm, nl_v)

        iota = lax.iota(jnp.int32, _LANES)

        def sb_body(qb, _):
            qo = qb * Q

            def level_body(l, _):
                nl = nl_v[pl.ds(l * _LANES, _LANES)]
                lR = l * (_T // 8)

                def idx_body(g, _):
                    o = g * _LANES
                    pid = (qo + o + iota) * 3
                    xn0 = plsc.load_gather(xs_v, [pid]) * nl
                    xn1 = plsc.load_gather(xs_v, [pid + 1]) * nl
                    xn2 = plsc.load_gather(xs_v, [pid + 2]) * nl
                    lb0 = xn0.astype(jnp.int32)
                    lb1 = xn1.astype(jnp.int32)
                    lb2 = xn2.astype(jnp.int32)
                    fr_v[pl.ds(o, _LANES)] = xn0 - lb0.astype(jnp.float32)
                    fr_v[pl.ds(Q + o, _LANES)] = xn1 - lb1.astype(jnp.float32)
                    fr_v[pl.ds(2 * Q + o, _LANES)] = xn2 - lb2.astype(jnp.float32)
                    a1 = lb1 * _PI1
                    a2 = lb2 * _PI2
                    b0 = lb0 + 1
                    b1 = a1 + _PI1
                    b2 = a2 + _PI2
                    for c in range(8):
                        h0 = b0 if (c >> 2) & 1 else lb0
                        h1 = b1 if (c >> 1) & 1 else a1
                        h2 = b2 if c & 1 else a2
                        h = (h0 ^ h1 ^ h2) & _MASK
                        co = pl.ds(c * Q + o, _LANES)
                        idx_v[co] = (h >> 3) + lR
                        col_v[co] = (h & 7) << 1

                lax.fori_loop(0, GQ, idx_body, None)

                def dma_body(jj, _):
                    hs = []
                    for j2 in range(KOUT):
                        ch = jj * KOUT + j2
                        hs.append(pltpu.async_copy(
                            table_hbm.at[idx_v.at[pl.ds(ch * CH, CH)]],
                            rows_v.at[pl.ds(ch * CH, CH)], sem))
                    for h in hs:
                        h.wait()

                lax.fori_loop(0, NCH // KOUT, dma_body, None)

                def interp_body(g, _):
                    o = g * _LANES
                    f0 = fr_v[pl.ds(o, _LANES)]
                    f1 = fr_v[pl.ds(Q + o, _LANES)]
                    f2 = fr_v[pl.ds(2 * Q + o, _LANES)]
                    g0 = 1.0 - f0
                    g1 = 1.0 - f1
                    g2 = 1.0 - f2
                    pair = (g0 * g1, g0 * f1, f0 * g1, f0 * f1)
                    rid0 = o + iota
                    acc0 = jnp.zeros((_LANES,), jnp.float32)
                    acc1 = jnp.zeros((_LANES,), jnp.float32)
                    for c in range(8):
                        w = pair[c >> 1] * (f2 if c & 1 else g2)
                        rid = rid0 + c * Q
                        col = col_v[pl.ds(c * Q + o, _LANES)]
                        e0 = plsc.load_gather(rows_v, [rid, col])
                        e1 = plsc.load_gather(rows_v, [rid, col + 1])
                        acc0 = acc0 + w * e0
                        acc1 = acc1 + w * e1
                    lvl2 = jnp.full((_LANES,), 2 * l, jnp.int32)
                    plsc.store_scatter(o_v, [rid0, lvl2], acc0)
                    plsc.store_scatter(o_v, [rid0, lvl2 + 1], acc1)

                lax.fori_loop(0, GQ, interp_body, None)

            lax.fori_loop(0, _L, level_body, None)
            pltpu.sync_copy(o_v, out_hbm.at[pl.ds(base + qo, Q), :])

        lax.fori_loop(0, SB, sb_body, None)

    return enc


def kernel(x, hashtable):
    N = x.shape[0]
    b = jnp.exp(jnp.log(_N_MAX / _N_MIN) / (_L - 1))
    n_levels = jnp.floor(_N_MIN * b ** jnp.arange(_L))
    nl_b = jnp.broadcast_to(n_levels[:, None].astype(jnp.float32),
                            (_L, _LANES)).reshape(-1)
    xf = x.reshape(-1)
    table = hashtable.reshape(_L * _T * _F // _RW, _RW)
    return _build(N)(xf, nl_b, table)


# speedup vs baseline: 9.5882x; 9.5882x over previous
import functools

import jax
import jax.numpy as jnp
from jax import lax
from jax.experimental import pallas as pl
from jax.experimental.pallas import tpu as pltpu
from jax.experimental.pallas import tpu_sc as plsc

_L = 16
_T = 2 ** 19
_F = 2
_N_MIN = 16
_N_MAX = 4096
_MASK = _T - 1
_PI1 = -1640531535
_PI2 = 805459861

_LANES = 16
_NC = 2
_NS = 16
_NW = _NC * _NS


def _build(N):
    P = N // _NW
    SB = 4
    Q = P // SB
    GQ = Q // _LANES
    NIQ = 8 * Q
    CH = 128
    NCH = 2 * NIQ // CH
    KOUT = 8

    mesh = plsc.VectorSubcoreMesh(core_axis_name="c", subcore_axis_name="s")

    @functools.partial(
        pl.kernel,
        out_type=jax.ShapeDtypeStruct((N, 2 * _L), jnp.float32),
        mesh=mesh,
        compiler_params=pltpu.CompilerParams(needs_layout_passes=False,
                                             use_tc_tiling_on_sc=False),
        scratch_types=[
            pltpu.VMEM((3 * P,), jnp.float32),
            pltpu.VMEM((3 * Q,), jnp.float32),
            pltpu.VMEM((_L * _LANES,), jnp.float32),
            pltpu.VMEM((2 * NIQ,), jnp.int32),
            pltpu.VMEM((2 * NIQ,), jnp.float32),
            pltpu.VMEM((Q, 2 * _L), jnp.float32),
            pltpu.SemaphoreType.DMA,
        ],
    )
    def enc(x_hbm, nl_hbm, table_hbm, out_hbm,
            xs_v, fr_v, nl_v, idx_v, e_v, o_v, sem):
        wid = lax.axis_index("s") * _NC + lax.axis_index("c")
        base = wid * P
        pltpu.sync_copy(x_hbm.at[pl.ds(3 * base, 3 * P)], xs_v)
        pltpu.sync_copy(nl_hbm, nl_v)

        iota = lax.iota(jnp.int32, _LANES)

        def sb_body(qb, _):
            qo = qb * Q

            def level_body(l, _):
                nl = nl_v[pl.ds(l * _LANES, _LANES)]
                lM = l * (_T * _F)

                def idx_body(g, _):
                    o = g * _LANES
                    pid = (qo + o + iota) * 3
                    xn0 = plsc.load_gather(xs_v, [pid]) * nl
                    xn1 = plsc.load_gather(xs_v, [pid + 1]) * nl
                    xn2 = plsc.load_gather(xs_v, [pid + 2]) * nl
                    lb0 = xn0.astype(jnp.int32)
                    lb1 = xn1.astype(jnp.int32)
                    lb2 = xn2.astype(jnp.int32)
                    fr_v[pl.ds(o, _LANES)] = xn0 - lb0.astype(jnp.float32)
                    fr_v[pl.ds(Q + o, _LANES)] = xn1 - lb1.astype(jnp.float32)
                    fr_v[pl.ds(2 * Q + o, _LANES)] = xn2 - lb2.astype(jnp.float32)
                    a1 = lb1 * _PI1
                    a2 = lb2 * _PI2
                    b0 = lb0 + 1
                    b1 = a1 + _PI1
                    b2 = a2 + _PI2
                    for c in range(8):
                        h0 = b0 if (c >> 2) & 1 else lb0
                        h1 = b1 if (c >> 1) & 1 else a1
                        h2 = b2 if c & 1 else a2
                        h = (h0 ^ h1 ^ h2) & _MASK
                        p0 = lM + ((h >> 7) << 8) + (h & 127)
                        co = c * Q + o
                        idx_v[pl.ds(co, _LANES)] = p0
                        idx_v[pl.ds(NIQ + co, _LANES)] = p0 + 128

                lax.fori_loop(0, GQ, idx_body, None)

                def dma_body(jj, _):
                    hs = []
                    for j2 in range(KOUT):
                        ch = jj * KOUT + j2
                        hs.append(pltpu.async_copy(
                            table_hbm.at[idx_v.at[pl.ds(ch * CH, CH)]],
                            e_v.at[pl.ds(ch * CH, CH)], sem))
                    for h in hs:
                        h.wait()

                lax.fori_loop(0, NCH // KOUT, dma_body, None)

                def interp_body(g, _):
                    o = g * _LANES
                    f0 = fr_v[pl.ds(o, _LANES)]
                    f1 = fr_v[pl.ds(Q + o, _LANES)]
                    f2 = fr_v[pl.ds(2 * Q + o, _LANES)]
                    g0 = 1.0 - f0
                    g1 = 1.0 - f1
                    g2 = 1.0 - f2
                    pair = (g0 * g1, g0 * f1, f0 * g1, f0 * f1)
                    acc0 = jnp.zeros((_LANES,), jnp.float32)
                    acc1 = jnp.zeros((_LANES,), jnp.float32)
                    for c in range(8):
                        w = pair[c >> 1] * (f2 if c & 1 else g2)
                        co = c * Q + o
                        acc0 = acc0 + w * e_v[pl.ds(co, _LANES)]
                        acc1 = acc1 + w * e_v[pl.ds(NIQ + co, _LANES)]
                    rid0 = o + iota
                    lvl2 = jnp.full((_LANES,), 2 * l, jnp.int32)
                    plsc.store_scatter(o_v, [rid0, lvl2], acc0)
                    plsc.store_scatter(o_v, [rid0, lvl2 + 1], acc1)

                lax.fori_loop(0, GQ, interp_body, None)

            lax.fori_loop(0, _L, level_body, None)
            pltpu.sync_copy(o_v, out_hbm.at[pl.ds(base + qo, Q), :])

        lax.fori_loop(0, SB, sb_body, None)

    return enc


def kernel(x, hashtable):
    N = x.shape[0]
    b = jnp.exp(jnp.log(_N_MAX / _N_MIN) / (_L - 1))
    n_levels = jnp.floor(_N_MIN * b ** jnp.arange(_L))
    nl_b = jnp.broadcast_to(n_levels[:, None].astype(jnp.float32),
                            (_L, _LANES)).reshape(-1)
    xf = x.reshape(-1)
    traw = hashtable.reshape(_L, _T // 128, 128, _F)
    traw = traw.transpose(0, 1, 3, 2).reshape(-1)
    return _build(N)(xf, nl_b, traw)


# speedup vs baseline: 12.8334x; 1.3384x over previous
import functools

import jax
import jax.numpy as jnp
from jax import lax
from jax.experimental import pallas as pl
from jax.experimental.pallas import tpu as pltpu
from jax.experimental.pallas import tpu_sc as plsc

_L = 16
_T = 2 ** 19
_F = 2
_N_MIN = 16
_N_MAX = 4096
_MASK = _T - 1
_PI1 = -1640531535
_PI2 = 805459861

_LANES = 16
_NC = 2
_NS = 16
_NW = _NC * _NS


def _build(N):
    P = N // _NW
    SB = 4
    Q = P // SB
    GQ = Q // _LANES
    NIQ = 8 * Q
    CH = 128
    NCH = 2 * NIQ // CH
    KOUT = 8

    mesh = plsc.VectorSubcoreMesh(core_axis_name="c", subcore_axis_name="s")

    @functools.partial(
        pl.kernel,
        out_type=jax.ShapeDtypeStruct((N, 2 * _L), jnp.float32),
        mesh=mesh,
        compiler_params=pltpu.CompilerParams(needs_layout_passes=False,
                                             use_tc_tiling_on_sc=False),
        scratch_types=[
            pltpu.VMEM((3 * P,), jnp.float32),
            pltpu.VMEM((3 * Q,), jnp.float32),
            pltpu.VMEM((_L * _LANES,), jnp.float32),
            pltpu.VMEM((2 * NIQ,), jnp.int32),
            pltpu.VMEM((2 * NIQ,), jnp.float32),
            pltpu.VMEM((Q, 2 * _L), jnp.float32),
            pltpu.SemaphoreType.DMA,
        ],
    )
    def enc(x_hbm, nl_hbm, table_hbm, out_hbm,
            xs_v, fr_v, nl_v, idx_v, e_v, o_v, sem):
        wid = lax.axis_index("s") * _NC + lax.axis_index("c")
        base = wid * P
        pltpu.sync_copy(x_hbm.at[pl.ds(3 * base, 3 * P)], xs_v)
        pltpu.sync_copy(nl_hbm, nl_v)

        iota = lax.iota(jnp.int32, _LANES)

        def sb_body(qb, _):
            qo = qb * Q

            def level_body(l, _):
                nl = nl_v[pl.ds(l * _LANES, _LANES)]
                lM = l * (_T * _F)

                def idx_body(g, _):
                    o = g * _LANES
                    pid = (qo + o + iota) * 3
                    xn0 = plsc.load_gather(xs_v, [pid]) * nl
                    xn1 = plsc.load_gather(xs_v, [pid + 1]) * nl
                    xn2 = plsc.load_gather(xs_v, [pid + 2]) * nl
                    lb0 = xn0.astype(jnp.int32)
                    lb1 = xn1.astype(jnp.int32)
                    lb2 = xn2.astype(jnp.int32)
                    fr_v[pl.ds(o, _LANES)] = xn0 - lb0.astype(jnp.float32)
                    fr_v[pl.ds(Q + o, _LANES)] = xn1 - lb1.astype(jnp.float32)
                    fr_v[pl.ds(2 * Q + o, _LANES)] = xn2 - lb2.astype(jnp.float32)
                    a1 = lb1 * _PI1
                    a2 = lb2 * _PI2
                    b0 = lb0 + 1
                    b1 = a1 + _PI1
                    b2 = a2 + _PI2
                    for c in range(8):
                        h0 = b0 if (c >> 2) & 1 else lb0
                        h1 = b1 if (c >> 1) & 1 else a1
                        h2 = b2 if c & 1 else a2
                        h = (h0 ^ h1 ^ h2) & _MASK
                        p0 = lM + ((h >> 7) << 8) + (h & 127)
                        co = c * Q + o
                        idx_v[pl.ds(co, _LANES)] = p0
                        idx_v[pl.ds(NIQ + co, _LANES)] = p0 + 128

                lax.fori_loop(0, GQ, idx_body, None)

                pltpu.async_copy(table_hbm.at[idx_v], e_v, sem).wait()

                def interp_body(g, _):
                    o = g * _LANES
                    f0 = fr_v[pl.ds(o, _LANES)]
                    f1 = fr_v[pl.ds(Q + o, _LANES)]
                    f2 = fr_v[pl.ds(2 * Q + o, _LANES)]
                    g0 = 1.0 - f0
                    g1 = 1.0 - f1
                    g2 = 1.0 - f2
                    pair = (g0 * g1, g0 * f1, f0 * g1, f0 * f1)
                    acc0 = jnp.zeros((_LANES,), jnp.float32)
                    acc1 = jnp.zeros((_LANES,), jnp.float32)
                    for c in range(8):
                        w = pair[c >> 1] * (f2 if c & 1 else g2)
                        co = c * Q + o
                        acc0 = acc0 + w * e_v[pl.ds(co, _LANES)]
                        acc1 = acc1 + w * e_v[pl.ds(NIQ + co, _LANES)]
                    rid0 = o + iota
                    lvl2 = jnp.full((_LANES,), 2 * l, jnp.int32)
                    plsc.store_scatter(o_v, [rid0, lvl2], acc0)
                    plsc.store_scatter(o_v, [rid0, lvl2 + 1], acc1)

                lax.fori_loop(0, GQ, interp_body, None)

            lax.fori_loop(0, _L, level_body, None)
            pltpu.sync_copy(o_v, out_hbm.at[pl.ds(base + qo, Q), :])

        lax.fori_loop(0, SB, sb_body, None)

    return enc


def kernel(x, hashtable):
    N = x.shape[0]
    b = jnp.exp(jnp.log(_N_MAX / _N_MIN) / (_L - 1))
    n_levels = jnp.floor(_N_MIN * b ** jnp.arange(_L))
    nl_b = jnp.broadcast_to(n_levels[:, None].astype(jnp.float32),
                            (_L, _LANES)).reshape(-1)
    xf = x.reshape(-1)
    traw = hashtable.reshape(_L, _T // 128, 128, _F)
    traw = traw.transpose(0, 1, 3, 2).reshape(-1)
    return _build(N)(xf, nl_b, traw)


# speedup vs baseline: 13.7575x; 1.0720x over previous
import functools

import jax
import jax.numpy as jnp
from jax import lax
from jax.experimental import pallas as pl
from jax.experimental.pallas import tpu as pltpu
from jax.experimental.pallas import tpu_sc as plsc

_L = 16
_T = 2 ** 19
_F = 2
_N_MIN = 16
_N_MAX = 4096
_MASK = _T - 1
_PI1 = -1640531535
_PI2 = 805459861

_LANES = 16
_NC = 2
_NS = 16
_NW = _NC * _NS


def _build(N):
    P = N // _NW
    SB = 4
    Q = P // SB
    GQ = Q // _LANES
    NIQ = 8 * Q
    CH = 128
    NCH = 2 * NIQ // CH
    KOUT = 8

    mesh = plsc.VectorSubcoreMesh(core_axis_name="c", subcore_axis_name="s")

    @functools.partial(
        pl.kernel,
        out_type=jax.ShapeDtypeStruct((N, 2 * _L), jnp.float32),
        mesh=mesh,
        compiler_params=pltpu.CompilerParams(needs_layout_passes=False,
                                             use_tc_tiling_on_sc=False),
        scratch_types=[
            pltpu.VMEM((3 * P,), jnp.float32),
            pltpu.VMEM((2 * 3 * Q,), jnp.float32),
            pltpu.VMEM((_L * _LANES,), jnp.float32),
            pltpu.VMEM((2 * 2 * NIQ,), jnp.int32),
            pltpu.VMEM((2 * 2 * NIQ,), jnp.float32),
            pltpu.VMEM((Q, 2 * _L), jnp.float32),
            pltpu.SemaphoreType.DMA,
        ],
    )
    def enc(x_hbm, nl_hbm, table_hbm, out_hbm,
            xs_v, fr_v, nl_v, idx_v, e_v, o_v, sem):
        wid = lax.axis_index("s") * _NC + lax.axis_index("c")
        base = wid * P
        pltpu.sync_copy(x_hbm.at[pl.ds(3 * base, 3 * P)], xs_v)
        pltpu.sync_copy(nl_hbm, nl_v)

        iota = lax.iota(jnp.int32, _LANES)

        def sb_body(qb, _):
            qo = qb * Q

            def compute_idx(l, ib, fb):
                nl = nl_v[pl.ds(l * _LANES, _LANES)]
                lM = l * (_T * _F)

                def idx_body(g, _):
                    o = g * _LANES
                    pid = (qo + o + iota) * 3
                    xn0 = plsc.load_gather(xs_v, [pid]) * nl
                    xn1 = plsc.load_gather(xs_v, [pid + 1]) * nl
                    xn2 = plsc.load_gather(xs_v, [pid + 2]) * nl
                    lb0 = xn0.astype(jnp.int32)
                    lb1 = xn1.astype(jnp.int32)
                    lb2 = xn2.astype(jnp.int32)
                    fr_v[pl.ds(fb + o, _LANES)] = xn0 - lb0.astype(jnp.float32)
                    fr_v[pl.ds(fb + Q + o, _LANES)] = xn1 - lb1.astype(jnp.float32)
                    fr_v[pl.ds(fb + 2 * Q + o, _LANES)] = xn2 - lb2.astype(jnp.float32)
                    a1 = lb1 * _PI1
                    a2 = lb2 * _PI2
                    b0 = lb0 + 1
                    b1 = a1 + _PI1
                    b2 = a2 + _PI2
                    for c in range(8):
                        h0 = b0 if (c >> 2) & 1 else lb0
                        h1 = b1 if (c >> 1) & 1 else a1
                        h2 = b2 if c & 1 else a2
                        h = (h0 ^ h1 ^ h2) & _MASK
                        p0 = lM + ((h >> 7) << 8) + (h & 127)
                        co = c * Q + o
                        idx_v[pl.ds(ib + co, _LANES)] = p0
                        idx_v[pl.ds(ib + NIQ + co, _LANES)] = p0 + 128

                lax.fori_loop(0, GQ, idx_body, None)

            def fire(ib):
                return pltpu.async_copy(
                    table_hbm.at[idx_v.at[pl.ds(ib, 2 * NIQ)]],
                    e_v.at[pl.ds(ib, 2 * NIQ)], sem)

            def interp(l, ib, fb):
                def interp_body(g, _):
                    o = g * _LANES
                    f0 = fr_v[pl.ds(fb + o, _LANES)]
                    f1 = fr_v[pl.ds(fb + Q + o, _LANES)]
                    f2 = fr_v[pl.ds(fb + 2 * Q + o, _LANES)]
                    g0 = 1.0 - f0
                    g1 = 1.0 - f1
                    g2 = 1.0 - f2
                    pair = (g0 * g1, g0 * f1, f0 * g1, f0 * f1)
                    acc0 = jnp.zeros((_LANES,), jnp.float32)
                    acc1 = jnp.zeros((_LANES,), jnp.float32)
                    for c in range(8):
                        w = pair[c >> 1] * (f2 if c & 1 else g2)
                        co = c * Q + o
                        acc0 = acc0 + w * e_v[pl.ds(ib + co, _LANES)]
                        acc1 = acc1 + w * e_v[pl.ds(ib + NIQ + co, _LANES)]
                    rid0 = o + iota
                    lvl2 = jnp.full((_LANES,), 2 * l, jnp.int32)
                    plsc.store_scatter(o_v, [rid0, lvl2], acc0)
                    plsc.store_scatter(o_v, [rid0, lvl2 + 1], acc1)

                lax.fori_loop(0, GQ, interp_body, None)

            def pl_body(l, _):
                buf = l & 1
                ib = buf * 2 * NIQ
                fb = buf * 3 * Q
                pb = (1 - buf) * 2 * NIQ
                qb_ = (1 - buf) * 3 * Q

                @pl.when(l < _L)
                def _():
                    compute_idx(l, ib, fb)
                    h = fire(ib)

                    @pl.when(l > 0)
                    def _():
                        interp(l - 1, pb, qb_)

                    h.wait()

                @pl.when(l == _L)
                def _():
                    interp(_L - 1, pb, qb_)

            lax.fori_loop(0, _L + 1, pl_body, None)
            pltpu.sync_copy(o_v, out_hbm.at[pl.ds(base + qo, Q), :])

        lax.fori_loop(0, SB, sb_body, None)

    return enc


def kernel(x, hashtable):
    N = x.shape[0]
    b = jnp.exp(jnp.log(_N_MAX / _N_MIN) / (_L - 1))
    n_levels = jnp.floor(_N_MIN * b ** jnp.arange(_L))
    nl_b = jnp.broadcast_to(n_levels[:, None].astype(jnp.float32),
                            (_L, _LANES)).reshape(-1)
    xf = x.reshape(-1)
    traw = hashtable.reshape(_L, _T // 128, 128, _F)
    traw = traw.transpose(0, 1, 3, 2).reshape(-1)
    return _build(N)(xf, nl_b, traw)


# speedup vs baseline: 14.5150x; 1.0551x over previous
import functools

import jax
import jax.numpy as jnp
from jax import lax
from jax.experimental import pallas as pl
from jax.experimental.pallas import tpu as pltpu
from jax.experimental.pallas import tpu_sc as plsc

_L = 16
_T = 2 ** 19
_F = 2
_N_MIN = 16
_N_MAX = 4096
_MASK = _T - 1
_PI1 = -1640531535
_PI2 = 805459861

_LANES = 16
_NC = 2
_NS = 16
_NW = _NC * _NS


def _build(N):
    P = N // _NW
    SB = 4
    Q = P // SB
    GQ = Q // _LANES
    NIQ = 8 * Q
    CH = 128
    NCH = 2 * NIQ // CH
    KOUT = 8

    mesh = plsc.VectorSubcoreMesh(core_axis_name="c", subcore_axis_name="s")

    @functools.partial(
        pl.kernel,
        out_type=jax.ShapeDtypeStruct((N, 2 * _L), jnp.float32),
        mesh=mesh,
        compiler_params=pltpu.CompilerParams(needs_layout_passes=False,
                                             use_tc_tiling_on_sc=False),
        scratch_types=[
            pltpu.VMEM((3 * P,), jnp.float32),
            pltpu.VMEM((2 * 3 * Q,), jnp.float32),
            pltpu.VMEM((_L * _LANES,), jnp.float32),
            pltpu.VMEM((2 * 2 * NIQ,), jnp.int32),
            pltpu.VMEM((2 * 2 * NIQ,), jnp.float32),
            pltpu.VMEM((Q, 2 * _L), jnp.float32),
            pltpu.SemaphoreType.DMA,
        ],
    )
    def enc(x_hbm, nl_hbm, table_hbm, out_hbm,
            xs_v, fr_v, nl_v, idx_v, e_v, o_v, sem):
        wid = lax.axis_index("s") * _NC + lax.axis_index("c")
        base = wid * P
        pltpu.sync_copy(x_hbm.at[pl.ds(3 * base, 3 * P)], xs_v)
        pltpu.sync_copy(nl_hbm, nl_v)

        iota = lax.iota(jnp.int32, _LANES)

        def sb_body(qb, _):
            qo = qb * Q

            def idx_group(g, nl, lM, ib, fb):
                o = g * _LANES
                pid = (qo + o + iota) * 3
                xn0 = plsc.load_gather(xs_v, [pid]) * nl
                xn1 = plsc.load_gather(xs_v, [pid + 1]) * nl
                xn2 = plsc.load_gather(xs_v, [pid + 2]) * nl
                lb0 = xn0.astype(jnp.int32)
                lb1 = xn1.astype(jnp.int32)
                lb2 = xn2.astype(jnp.int32)
                fr_v[pl.ds(fb + o, _LANES)] = xn0 - lb0.astype(jnp.float32)
                fr_v[pl.ds(fb + Q + o, _LANES)] = xn1 - lb1.astype(jnp.float32)
                fr_v[pl.ds(fb + 2 * Q + o, _LANES)] = xn2 - lb2.astype(jnp.float32)
                a1 = lb1 * _PI1
                a2 = lb2 * _PI2
                b0 = lb0 + 1
                b1 = a1 + _PI1
                b2 = a2 + _PI2
                for c in range(8):
                    h0 = b0 if (c >> 2) & 1 else lb0
                    h1 = b1 if (c >> 1) & 1 else a1
                    h2 = b2 if c & 1 else a2
                    h = (h0 ^ h1 ^ h2) & _MASK
                    p0 = lM + ((h >> 7) << 8) + (h & 127)
                    co = c * Q + o
                    idx_v[pl.ds(ib + co, _LANES)] = p0
                    idx_v[pl.ds(ib + NIQ + co, _LANES)] = p0 + 128

            def interp_group(g, lvl2, ib, fb):
                o = g * _LANES
                f0 = fr_v[pl.ds(fb + o, _LANES)]
                f1 = fr_v[pl.ds(fb + Q + o, _LANES)]
                f2 = fr_v[pl.ds(fb + 2 * Q + o, _LANES)]
                g0 = 1.0 - f0
                g1 = 1.0 - f1
                g2 = 1.0 - f2
                pair = (g0 * g1, g0 * f1, f0 * g1, f0 * f1)
                acc0 = jnp.zeros((_LANES,), jnp.float32)
                acc1 = jnp.zeros((_LANES,), jnp.float32)
                for c in range(8):
                    w = pair[c >> 1] * (f2 if c & 1 else g2)
                    co = c * Q + o
                    acc0 = acc0 + w * e_v[pl.ds(ib + co, _LANES)]
                    acc1 = acc1 + w * e_v[pl.ds(ib + NIQ + co, _LANES)]
                rid0 = o + iota
                plsc.store_scatter(o_v, [rid0, lvl2], acc0)
                plsc.store_scatter(o_v, [rid0, lvl2 + 1], acc1)

            def compute_idx(l, ib, fb):
                nl = nl_v[pl.ds(l * _LANES, _LANES)]
                lM = l * (_T * _F)
                lax.fori_loop(
                    0, GQ, lambda g, _: idx_group(g, nl, lM, ib, fb), None)

            def interp(l, ib, fb):
                lvl2 = jnp.full((_LANES,), 2 * l, jnp.int32)
                lax.fori_loop(
                    0, GQ, lambda g, _: interp_group(g, lvl2, ib, fb), None)

            def fire(ib):
                return pltpu.async_copy(
                    table_hbm.at[idx_v.at[pl.ds(ib, 2 * NIQ)]],
                    e_v.at[pl.ds(ib, 2 * NIQ)], sem)

            compute_idx(0, 0, 0)

            def pl_body(l, _):
                buf = l & 1
                ib = buf * 2 * NIQ
                fb = buf * 3 * Q
                ibn = (1 - buf) * 2 * NIQ
                fbn = (1 - buf) * 3 * Q
                h = fire(ib)

                @pl.when(l == 0)
                def _():
                    compute_idx(1, ibn, fbn)

                @pl.when(jnp.logical_and(l > 0, l < _L - 1))
                def _():
                    nl = nl_v[pl.ds((l + 1) * _LANES, _LANES)]
                    lM = (l + 1) * (_T * _F)
                    lvl2 = jnp.full((_LANES,), 2 * (l - 1), jnp.int32)

                    def fused(g, _):
                        interp_group(g, lvl2, ibn, fbn)
                        idx_group(g, nl, lM, ibn, fbn)

                    lax.fori_loop(0, GQ, fused, None)

                @pl.when(l == _L - 1)
                def _():
                    interp(l - 1, ibn, fbn)

                h.wait()

            lax.fori_loop(0, _L, pl_body, None)
            interp(_L - 1, 2 * NIQ, 3 * Q)
            pltpu.sync_copy(o_v, out_hbm.at[pl.ds(base + qo, Q), :])

        lax.fori_loop(0, SB, sb_body, None)

    return enc


def kernel(x, hashtable):
    N = x.shape[0]
    b = jnp.exp(jnp.log(_N_MAX / _N_MIN) / (_L - 1))
    n_levels = jnp.floor(_N_MIN * b ** jnp.arange(_L))
    nl_b = jnp.broadcast_to(n_levels[:, None].astype(jnp.float32),
                            (_L, _LANES)).reshape(-1)
    xf = x.reshape(-1)
    traw = hashtable.reshape(_L, _T // 128, 128, _F)
    traw = traw.transpose(0, 1, 3, 2).reshape(-1)
    return _build(N)(xf, nl_b, traw)


# speedup vs baseline: 14.5671x; 1.0036x over previous
import functools

import jax
import jax.numpy as jnp
from jax import lax
from jax.experimental import pallas as pl
from jax.experimental.pallas import tpu as pltpu
from jax.experimental.pallas import tpu_sc as plsc

_L = 16
_T = 2 ** 19
_F = 2
_N_MIN = 16
_N_MAX = 4096
_MASK = _T - 1
_PI1 = -1640531535
_PI2 = 805459861

_LANES = 16
_NC = 2
_NS = 16
_NW = _NC * _NS


def _build(N):
    P = N // _NW
    SB = 2
    Q = P // SB
    GQ = Q // _LANES
    NIQ = 8 * Q
    CH = 128
    NCH = 2 * NIQ // CH
    KOUT = 8

    mesh = plsc.VectorSubcoreMesh(core_axis_name="c", subcore_axis_name="s")

    @functools.partial(
        pl.kernel,
        out_type=jax.ShapeDtypeStruct((N, 2 * _L), jnp.float32),
        mesh=mesh,
        compiler_params=pltpu.CompilerParams(needs_layout_passes=False,
                                             use_tc_tiling_on_sc=False),
        scratch_types=[
            pltpu.VMEM((3 * P,), jnp.float32),
            pltpu.VMEM((2 * 3 * Q,), jnp.float32),
            pltpu.VMEM((_L * _LANES,), jnp.float32),
            pltpu.VMEM((2 * 2 * NIQ,), jnp.int32),
            pltpu.VMEM((2 * 2 * NIQ,), jnp.float32),
            pltpu.VMEM((Q, 2 * _L), jnp.float32),
            pltpu.SemaphoreType.DMA,
        ],
    )
    def enc(x_hbm, nl_hbm, table_hbm, out_hbm,
            xs_v, fr_v, nl_v, idx_v, e_v, o_v, sem):
        wid = lax.axis_index("s") * _NC + lax.axis_index("c")
        base = wid * P
        pltpu.sync_copy(x_hbm.at[pl.ds(3 * base, 3 * P)], xs_v)
        pltpu.sync_copy(nl_hbm, nl_v)

        iota = lax.iota(jnp.int32, _LANES)

        def sb_body(qb, _):
            qo = qb * Q

            def idx_group(g, nl, lM, ib, fb):
                o = g * _LANES
                pid = (qo + o + iota) * 3
                xn0 = plsc.load_gather(xs_v, [pid]) * nl
                xn1 = plsc.load_gather(xs_v, [pid + 1]) * nl
                xn2 = plsc.load_gather(xs_v, [pid + 2]) * nl
                lb0 = xn0.astype(jnp.int32)
                lb1 = xn1.astype(jnp.int32)
                lb2 = xn2.astype(jnp.int32)
                fr_v[pl.ds(fb + o, _LANES)] = xn0 - lb0.astype(jnp.float32)
                fr_v[pl.ds(fb + Q + o, _LANES)] = xn1 - lb1.astype(jnp.float32)
                fr_v[pl.ds(fb + 2 * Q + o, _LANES)] = xn2 - lb2.astype(jnp.float32)
                a1 = lb1 * _PI1
                a2 = lb2 * _PI2
                b0 = lb0 + 1
                b1 = a1 + _PI1
                b2 = a2 + _PI2
                for c in range(8):
                    h0 = b0 if (c >> 2) & 1 else lb0
                    h1 = b1 if (c >> 1) & 1 else a1
                    h2 = b2 if c & 1 else a2
                    h = (h0 ^ h1 ^ h2) & _MASK
                    p0 = lM + ((h >> 7) << 8) + (h & 127)
                    co = c * Q + o
                    idx_v[pl.ds(ib + co, _LANES)] = p0
                    idx_v[pl.ds(ib + NIQ + co, _LANES)] = p0 + 128

            def interp_group(g, lvl2, ib, fb):
                o = g * _LANES
                f0 = fr_v[pl.ds(fb + o, _LANES)]
                f1 = fr_v[pl.ds(fb + Q + o, _LANES)]
                f2 = fr_v[pl.ds(fb + 2 * Q + o, _LANES)]
                g0 = 1.0 - f0
                g1 = 1.0 - f1
                g2 = 1.0 - f2
                pair = (g0 * g1, g0 * f1, f0 * g1, f0 * f1)
                acc0 = jnp.zeros((_LANES,), jnp.float32)
                acc1 = jnp.zeros((_LANES,), jnp.float32)
                for c in range(8):
                    w = pair[c >> 1] * (f2 if c & 1 else g2)
                    co = c * Q + o
                    acc0 = acc0 + w * e_v[pl.ds(ib + co, _LANES)]
                    acc1 = acc1 + w * e_v[pl.ds(ib + NIQ + co, _LANES)]
                rid0 = o + iota
                plsc.store_scatter(o_v, [rid0, lvl2], acc0)
                plsc.store_scatter(o_v, [rid0, lvl2 + 1], acc1)

            def compute_idx(l, ib, fb):
                nl = nl_v[pl.ds(l * _LANES, _LANES)]
                lM = l * (_T * _F)
                lax.fori_loop(
                    0, GQ, lambda g, _: idx_group(g, nl, lM, ib, fb), None)

            def interp(l, ib, fb):
                lvl2 = jnp.full((_LANES,), 2 * l, jnp.int32)
                lax.fori_loop(
                    0, GQ, lambda g, _: interp_group(g, lvl2, ib, fb), None)

            def fire(ib):
                return pltpu.async_copy(
                    table_hbm.at[idx_v.at[pl.ds(ib, 2 * NIQ)]],
                    e_v.at[pl.ds(ib, 2 * NIQ)], sem)

            compute_idx(0, 0, 0)

            def pl_body(l, _):
                buf = l & 1
                ib = buf * 2 * NIQ
                fb = buf * 3 * Q
                ibn = (1 - buf) * 2 * NIQ
                fbn = (1 - buf) * 3 * Q
                h = fire(ib)

                @pl.when(l == 0)
                def _():
                    compute_idx(1, ibn, fbn)

                @pl.when(jnp.logical_and(l > 0, l < _L - 1))
                def _():
                    nl = nl_v[pl.ds((l + 1) * _LANES, _LANES)]
                    lM = (l + 1) * (_T * _F)
                    lvl2 = jnp.full((_LANES,), 2 * (l - 1), jnp.int32)

                    def fused(g, _):
                        interp_group(g, lvl2, ibn, fbn)
                        idx_group(g, nl, lM, ibn, fbn)

                    lax.fori_loop(0, GQ, fused, None)

                @pl.when(l == _L - 1)
                def _():
                    interp(l - 1, ibn, fbn)

                h.wait()

            lax.fori_loop(0, _L, pl_body, None)
            interp(_L - 1, 2 * NIQ, 3 * Q)
            pltpu.sync_copy(o_v, out_hbm.at[pl.ds(base + qo, Q), :])

        lax.fori_loop(0, SB, sb_body, None)

    return enc


def kernel(x, hashtable):
    N = x.shape[0]
    b = jnp.exp(jnp.log(_N_MAX / _N_MIN) / (_L - 1))
    n_levels = jnp.floor(_N_MIN * b ** jnp.arange(_L))
    nl_b = jnp.broadcast_to(n_levels[:, None].astype(jnp.float32),
                            (_L, _LANES)).reshape(-1)
    xf = x.reshape(-1)
    traw = hashtable.reshape(_L, _T // 128, 128, _F)
    traw = traw.transpose(0, 1, 3, 2).reshape(-1)
    return _build(N)(xf, nl_b, traw)
